# baseline (speedup 1.0000x reference)
import functools

import jax
import jax.numpy as jnp
from jax import lax
from jax.experimental import pallas as pl
from jax.experimental.pallas import tpu as pltpu
from jax.experimental.pallas import tpu_sc as plsc

N_NODES = 10000
D = 128
E = 320000
NC = 2
NS = 16
NT = NC * NS
KC = 128
KT = 16
EPT = E // NT
CHF = (EPT - KT) // KC
NBUF = 2
G = CHF // NBUF
ZC = 80
NZ = N_NODES // ZC
NW = (NZ + NS - 1) // NS
L = 16


def _sc_segment_sum(src, dst, feat):
    mesh = plsc.VectorSubcoreMesh(core_axis_name="c", subcore_axis_name="s")

    @functools.partial(
        pl.kernel,
        out_type=(
            jax.ShapeDtypeStruct((NC, N_NODES, D), jnp.float32),
            jax.ShapeDtypeStruct((NC, N_NODES, D), jnp.float32),
        ),
        mesh=mesh,
        scratch_types=[
            pltpu.VMEM((EPT,), jnp.int32),
            pltpu.VMEM((NBUF, KC), jnp.int32),
            pltpu.VMEM((KT,), jnp.int32),
            pltpu.VMEM((NBUF, KC, D), jnp.float32),
            pltpu.SemaphoreType.DMA((NBUF,)),
            pltpu.SemaphoreType.DMA((NBUF,)),
            pltpu.SemaphoreType.DMA,
            pltpu.VMEM_SHARED((N_NODES, D), jnp.float32),
        ],
    )
    def sc_fn(src_hbm, dst_hbm, feat_hbm,
              sum_hbm, deg_hbm,
              srcall_v, dstk_v, dstt_v, rows_v, gsem, isem, wsem, acc_sh):
        c = lax.axis_index("c")
        s = lax.axis_index("s")
        wid = c * NS + s

        zvec = jnp.zeros((L,), jnp.float32)
        onevec = jnp.ones((L,), jnp.float32)

        def fill_slot0(vec, nrows):
            @pl.loop(0, nrows)
            def _(r):
                @pl.loop(0, D // L)
                def _(q):
                    rows_v.at[0, r, pl.ds(q * L, L)][...] = vec

        def stage(b, n):
            return rows_v.at[b, pl.ds(0, n), :]

        def zero_acc():
            @pl.loop(0, NW)
            def _(i):
                cid = i * NS + s

                @pl.when(cid < NZ)
                def _():
                    pltpu.async_copy(
                        stage(0, ZC), acc_sh.at[pl.ds(cid * ZC, ZC)], wsem)

            @pl.loop(0, NW)
            def _(i):
                @pl.when(i * NS + s < NZ)
                def _():
                    pltpu.make_async_copy(
                        stage(0, ZC), acc_sh.at[pl.ds(0, ZC)], wsem).wait()

        def write_acc(out_hbm):
            @pl.loop(0, NW)
            def _(i):
                cid = i * NS + s

                @pl.when(cid < NZ)
                def _():
                    base = cid * ZC
                    pltpu.async_copy(
                        acc_sh.at[pl.ds(base, ZC)],
                        out_hbm.at[c, pl.ds(base, ZC)], wsem)

            @pl.loop(0, NW)
            def _(i):
                @pl.when(i * NS + s < NZ)
                def _():
                    pltpu.make_async_copy(
                        acc_sh.at[pl.ds(0, ZC)], out_hbm.at[c, pl.ds(0, ZC)],
                        wsem).wait()

        def src_idx(j):
            return srcall_v.at[pl.ds(j * KC, KC)]

        def fire_dst_idx(j, b):
            pltpu.async_copy(
                dst_hbm.at[pl.ds(wid * EPT + j * KC, KC)], dstk_v.at[b],
                isem.at[b])

        def wait_dst_idx(b):
            pltpu.make_async_copy(
                dst_hbm.at[pl.ds(0, KC)], dstk_v.at[b], isem.at[b]).wait()

        def load_tail_dst():
            pltpu.sync_copy(
                dst_hbm.at[pl.ds(wid * EPT + CHF * KC, KT)], dstt_v)

        pltpu.sync_copy(src_hbm.at[pl.ds(wid * EPT, EPT)], srcall_v)

        fill_slot0(zvec, ZC)
        zero_acc()
        plsc.subcore_barrier()

        for b in range(NBUF):
            pltpu.async_copy(
                feat_hbm.at[src_idx(b)], rows_v.at[b], gsem.at[b])
            fire_dst_idx(b, b)

        @pl.loop(0, G)
        def _(g):
            j0 = g * NBUF
            for b in range(NBUF):
                pltpu.make_async_copy(
                    feat_hbm.at[pl.ds(0, KC)], rows_v.at[b],
                    gsem.at[b]).wait()
                wait_dst_idx(b)
                pltpu.sync_copy(rows_v.at[b], acc_sh.at[dstk_v.at[b]],
                                add=True)

                @pl.when(j0 + b + NBUF < CHF)
                def _():
                    pltpu.async_copy(
                        feat_hbm.at[src_idx(j0 + b + NBUF)],
                        rows_v.at[b], gsem.at[b])
                    fire_dst_idx(j0 + b + NBUF, b)

        load_tail_dst()
        pltpu.async_copy(
            feat_hbm.at[srcall_v.at[pl.ds(CHF * KC, KT)]],
            stage(0, KT), gsem.at[0]).wait()
        pltpu.sync_copy(stage(0, KT), acc_sh.at[dstt_v], add=True)

        plsc.subcore_barrier()
        write_acc(sum_hbm)
        plsc.subcore_barrier()

        fill_slot0(onevec, KC)
        ones_v = rows_v.at[0]

        for b in range(NBUF):
            fire_dst_idx(b, b)

        @pl.loop(0, G)
        def _(g):
            j0 = g * NBUF
            for b in range(NBUF):
                wait_dst_idx(b)
                pltpu.sync_copy(ones_v, acc_sh.at[dstk_v.at[b]], add=True)

                @pl.when(j0 + b + NBUF < CHF)
                def _():
                    fire_dst_idx(j0 + b + NBUF, b)

        load_tail_dst()
        pltpu.sync_copy(stage(0, KT), acc_sh.at[dstt_v], add=True)

        plsc.subcore_barrier()
        write_acc(deg_hbm)

    return sc_fn(src, dst, feat)


def _tc_combine(feat, parts, dparts, wst, wnt):
    B = 2000

    def body(feat_ref, p_ref, d_ref, wst_ref, wnt_ref, out_ref):
        ssum = p_ref[0] + p_ref[1]
        deg = (d_ref[0, :, 0:1] + d_ref[1, :, 0:1]) - (
            p_ref[0, :, 0:1] + p_ref[1, :, 0:1])
        h = ssum / jnp.maximum(deg, 1.0)
        out_ref[...] = (
            jnp.dot(feat_ref[...], wst_ref[...],
                    preferred_element_type=jnp.float32)
            + jnp.dot(h, wnt_ref[...], preferred_element_type=jnp.float32))

    return pl.pallas_call(
        body,
        grid=(N_NODES // B,),
        in_specs=[
            pl.BlockSpec((B, D), lambda i: (i, 0)),
            pl.BlockSpec((NC, B, D), lambda i: (0, i, 0)),
            pl.BlockSpec((NC, B, D), lambda i: (0, i, 0)),
            pl.BlockSpec((D, D), lambda i: (0, 0)),
            pl.BlockSpec((D, D), lambda i: (0, 0)),
        ],
        out_specs=pl.BlockSpec((B, D), lambda i: (i, 0)),
        out_shape=jax.ShapeDtypeStruct((N_NODES, D), jnp.float32),
    )(feat, parts, dparts, wst, wnt)


def kernel(feat, edge_index, W_self, W_neigh):
    src = edge_index[0].reshape(E)
    dst = edge_index[1].reshape(E)
    parts, dparts = _sc_segment_sum(src, dst, feat)
    return _tc_combine(feat, parts, dparts, W_self.T, W_neigh.T)


# speedup vs baseline: 1.0172x; 1.0172x over previous
import functools

import jax
import jax.numpy as jnp
from jax import lax
from jax.experimental import pallas as pl
from jax.experimental.pallas import tpu as pltpu
from jax.experimental.pallas import tpu_sc as plsc

N_NODES = 10000
D = 128
E = 320000
NC = 2
NS = 16
NT = NC * NS
KC = 128
KT = 16
EPT = E // NT
CHF = (EPT - KT) // KC
NBUF = 2
G = CHF // NBUF
ZC = 80
NZ = N_NODES // ZC
NW = (NZ + NS - 1) // NS
L = 16


def _sc_segment_sum(src, dst, feat):
    mesh = plsc.VectorSubcoreMesh(core_axis_name="c", subcore_axis_name="s")

    @functools.partial(
        pl.kernel,
        out_type=(
            jax.ShapeDtypeStruct((NC, N_NODES, D), jnp.float32),
            jax.ShapeDtypeStruct((NC, N_NODES, D), jnp.float32),
        ),
        mesh=mesh,
        scratch_types=[
            pltpu.VMEM((EPT,), jnp.int32),
            pltpu.VMEM((NBUF, KC), jnp.int32),
            pltpu.VMEM((KT,), jnp.int32),
            pltpu.VMEM((NBUF, KC, D), jnp.float32),
            pltpu.SemaphoreType.DMA((NBUF,)),
            pltpu.SemaphoreType.DMA((NBUF,)),
            pltpu.SemaphoreType.DMA,
            pltpu.VMEM_SHARED((N_NODES, D), jnp.float32),
        ],
    )
    def sc_fn(src_hbm, dst_hbm, feat_hbm,
              sum_hbm, deg_hbm,
              srcall_v, dstk_v, dstt_v, rows_v, gsem, isem, wsem, acc_sh):
        c = lax.axis_index("c")
        s = lax.axis_index("s")
        wid = c * NS + s

        zvec = jnp.zeros((L,), jnp.float32)
        onevec = jnp.ones((L,), jnp.float32)

        def fill_slot0(vec):
            @pl.loop(0, KC)
            def _(r):
                @pl.loop(0, D // L)
                def _(q):
                    rows_v.at[0, r, pl.ds(q * L, L)][...] = vec

        def stage(b, n):
            return rows_v.at[b, pl.ds(0, n), :]

        def zero_acc():
            @pl.loop(0, NW)
            def _(i):
                cid = i * NS + s

                @pl.when(cid < NZ)
                def _():
                    pltpu.async_copy(
                        stage(0, ZC), acc_sh.at[pl.ds(cid * ZC, ZC)], wsem)

            @pl.loop(0, NW)
            def _(i):
                @pl.when(i * NS + s < NZ)
                def _():
                    pltpu.make_async_copy(
                        stage(0, ZC), acc_sh.at[pl.ds(0, ZC)], wsem).wait()

        def write_acc(out_hbm):
            @pl.loop(0, NW, step=2)
            def _(i):
                for b in range(2):
                    ii = i + b
                    cid = ii * NS + s

                    @pl.when(cid < NZ)
                    def _():
                        @pl.when(ii >= 2)
                        def _():
                            pltpu.make_async_copy(
                                stage(b, ZC), out_hbm.at[c, pl.ds(0, ZC)],
                                wsem).wait()
                        base = cid * ZC
                        pltpu.sync_copy(acc_sh.at[pl.ds(base, ZC)],
                                        stage(b, ZC))
                        pltpu.async_copy(
                            stage(b, ZC), out_hbm.at[c, pl.ds(base, ZC)],
                            wsem)

            for b in range(2):
                pltpu.make_async_copy(
                    stage(b, ZC), out_hbm.at[c, pl.ds(0, ZC)], wsem).wait()

        def src_idx(j):
            return srcall_v.at[pl.ds(j * KC, KC)]

        def fire_dst_idx(j, b):
            pltpu.async_copy(
                dst_hbm.at[pl.ds(wid * EPT + j * KC, KC)], dstk_v.at[b],
                isem.at[b])

        def wait_dst_idx(b):
            pltpu.make_async_copy(
                dst_hbm.at[pl.ds(0, KC)], dstk_v.at[b], isem.at[b]).wait()

        def load_tail_dst():
            pltpu.sync_copy(
                dst_hbm.at[pl.ds(wid * EPT + CHF * KC, KT)], dstt_v)

        pltpu.sync_copy(src_hbm.at[pl.ds(wid * EPT, EPT)], srcall_v)

        fill_slot0(zvec)
        zero_acc()
        plsc.subcore_barrier()

        for b in range(NBUF):
            pltpu.async_copy(
                feat_hbm.at[src_idx(b)], rows_v.at[b], gsem.at[b])
            fire_dst_idx(b, b)

        @pl.loop(0, G)
        def _(g):
            j0 = g * NBUF
            for b in range(NBUF):
                pltpu.make_async_copy(
                    feat_hbm.at[pl.ds(0, KC)], rows_v.at[b],
                    gsem.at[b]).wait()
                wait_dst_idx(b)
                pltpu.sync_copy(rows_v.at[b], acc_sh.at[dstk_v.at[b]],
                                add=True)

                @pl.when(j0 + b + NBUF < CHF)
                def _():
                    pltpu.async_copy(
                        feat_hbm.at[src_idx(j0 + b + NBUF)],
                        rows_v.at[b], gsem.at[b])
                    fire_dst_idx(j0 + b + NBUF, b)

        load_tail_dst()
        pltpu.async_copy(
            feat_hbm.at[srcall_v.at[pl.ds(CHF * KC, KT)]],
            stage(0, KT), gsem.at[0]).wait()
        pltpu.sync_copy(stage(0, KT), acc_sh.at[dstt_v], add=True)

        plsc.subcore_barrier()
        write_acc(sum_hbm)
        plsc.subcore_barrier()

        fill_slot0(onevec)
        ones_v = rows_v.at[0]

        for b in range(NBUF):
            fire_dst_idx(b, b)

        @pl.loop(0, G)
        def _(g):
            j0 = g * NBUF
            for b in range(NBUF):
                wait_dst_idx(b)
                pltpu.sync_copy(ones_v, acc_sh.at[dstk_v.at[b]], add=True)

                @pl.when(j0 + b + NBUF < CHF)
                def _():
                    fire_dst_idx(j0 + b + NBUF, b)

        load_tail_dst()
        pltpu.sync_copy(stage(0, KT), acc_sh.at[dstt_v], add=True)

        plsc.subcore_barrier()
        write_acc(deg_hbm)

    return sc_fn(src, dst, feat)


def _tc_combine(feat, parts, dparts, wst, wnt):
    B = 2000

    def body(feat_ref, p_ref, d_ref, wst_ref, wnt_ref, out_ref):
        ssum = p_ref[0] + p_ref[1]
        deg = (d_ref[0, :, 0:1] + d_ref[1, :, 0:1]) - (
            p_ref[0, :, 0:1] + p_ref[1, :, 0:1])
        h = ssum / jnp.maximum(deg, 1.0)
        out_ref[...] = (
            jnp.dot(feat_ref[...], wst_ref[...],
                    preferred_element_type=jnp.float32)
            + jnp.dot(h, wnt_ref[...], preferred_element_type=jnp.float32))

    return pl.pallas_call(
        body,
        grid=(N_NODES // B,),
        in_specs=[
            pl.BlockSpec((B, D), lambda i: (i, 0)),
            pl.BlockSpec((NC, B, D), lambda i: (0, i, 0)),
            pl.BlockSpec((NC, B, D), lambda i: (0, i, 0)),
            pl.BlockSpec((D, D), lambda i: (0, 0)),
            pl.BlockSpec((D, D), lambda i: (0, 0)),
        ],
        out_specs=pl.BlockSpec((B, D), lambda i: (i, 0)),
        out_shape=jax.ShapeDtypeStruct((N_NODES, D), jnp.float32),
    )(feat, parts, dparts, wst, wnt)


def kernel(feat, edge_index, W_self, W_neigh):
    src = edge_index[0].reshape(E)
    dst = edge_index[1].reshape(E)
    parts, dparts = _sc_segment_sum(src, dst, feat)
    return _tc_combine(feat, parts, dparts, W_self.T, W_neigh.T)
